# Initial kernel scaffold; baseline (speedup 1.0000x reference)
import jax
import jax.numpy as jnp
from jax.experimental import pallas as pl


def kernel(x, adj, W1, att_src1, att_dst1, b1, W2, att_src2, att_dst2, b2):
    raise NotImplementedError("write your pallas kernel here")


# speedup vs baseline: 3739.6930x; 3739.6930x over previous
import functools

import jax
import jax.numpy as jnp
from jax.experimental import pallas as pl

_BJ = 256


def _attend(asrc_col, adT_row, adj_blk, feat):
    s = asrc_col + adT_row
    s = jnp.where(s > 0, s, 0.2 * s)
    s = jnp.where(adj_blk > 0, s, -jnp.inf)
    m = jnp.max(s, axis=0, keepdims=True)
    m = jnp.where(jnp.isfinite(m), m, 0.0)
    e = jnp.exp(s - m)
    d = jnp.sum(e, axis=0, keepdims=True)
    p = e * (1.0 / (d + 1e-16))
    return jax.lax.dot_general(
        p, feat, (((0,), (0,)), ((), ())), preferred_element_type=jnp.float32
    )


def _layer1_kern(heads, ch, x_ref, xb_ref, adj_ref, W1_ref, As_ref, Ad_ref,
                 b1_ref, out_ref):
    x = x_ref[:]
    W1 = W1_ref[:]
    hfull = jnp.dot(x, W1, preferred_element_type=jnp.float32)
    asrc = jnp.dot(hfull, As_ref[:], preferred_element_type=jnp.float32)
    hblk = jnp.dot(xb_ref[:], W1, preferred_element_type=jnp.float32)
    adT = jax.lax.dot_general(
        Ad_ref[:], hblk, (((0,), (1,)), ((), ())),
        preferred_element_type=jnp.float32)
    adj_blk = adj_ref[:]
    parts = []
    for h in range(heads):
        oh = _attend(asrc[:, h:h + 1], adT[h:h + 1, :], adj_blk,
                     hfull[:, h * ch:(h + 1) * ch])
        parts.append(oh)
    o = jnp.concatenate(parts, axis=1) + b1_ref[:]
    out_ref[:] = jnp.where(o > 0, o, jnp.exp(o) - 1.0)


def _layer2_kern(h1_ref, h1b_ref, adj_ref, W2_ref, as2_ref, ad2_ref, b2_ref,
                 out_ref):
    W2 = W2_ref[:]
    h2full = jnp.dot(h1_ref[:], W2, preferred_element_type=jnp.float32)
    asrc = jnp.dot(h2full, as2_ref[:], preferred_element_type=jnp.float32)
    h2blk = jnp.dot(h1b_ref[:], W2, preferred_element_type=jnp.float32)
    adT = jax.lax.dot_general(
        ad2_ref[:], h2blk, (((0,), (1,)), ((), ())),
        preferred_element_type=jnp.float32)
    out_ref[:] = _attend(asrc, adT, adj_ref[:], h2full) + b2_ref[:]


def kernel(x, adj, W1, att_src1, att_dst1, b1, W2, att_src2, att_dst2, b2):
    n, f_in = x.shape
    heads, ch = att_src1.shape
    nc = W2.shape[1]
    grid = (n // _BJ,)

    eye = jnp.eye(heads, dtype=jnp.float32)
    As_full = (eye[:, None, :] * att_src1[:, :, None]).reshape(heads * ch, heads)
    Ad_full = (eye[:, None, :] * att_dst1[:, :, None]).reshape(heads * ch, heads)

    full = lambda r, c: pl.BlockSpec((r, c), lambda j: (0, 0))
    colblk = lambda r: pl.BlockSpec((r, _BJ), lambda j: (0, j))
    rowblk = lambda c: pl.BlockSpec((_BJ, c), lambda j: (j, 0))

    h1 = pl.pallas_call(
        functools.partial(_layer1_kern, heads, ch),
        grid=grid,
        in_specs=[full(n, f_in), rowblk(f_in), colblk(n),
                  full(f_in, heads * ch), full(heads * ch, heads),
                  full(heads * ch, heads), full(1, heads * ch)],
        out_specs=rowblk(heads * ch),
        out_shape=jax.ShapeDtypeStruct((n, heads * ch), jnp.float32),
    )(x, x, adj, W1, As_full, Ad_full, b1.reshape(1, -1))

    out = pl.pallas_call(
        _layer2_kern,
        grid=grid,
        in_specs=[full(n, heads * ch), rowblk(heads * ch), colblk(n),
                  full(heads * ch, nc), full(nc, 1), full(nc, 1),
                  full(1, nc)],
        out_specs=rowblk(nc),
        out_shape=jax.ShapeDtypeStruct((n, nc), jnp.float32),
    )(h1, h1, adj, W2, att_src2.reshape(nc, 1), att_dst2.reshape(nc, 1),
      b2.reshape(1, -1))
    return out
